# Initial kernel scaffold; baseline (speedup 1.0000x reference)
import jax
import jax.numpy as jnp
from jax.experimental import pallas as pl


def kernel(drug_x, prot_x, d_feature, p_feature, W_denc, b_denc, W_penc, b_penc, W_d, b_d, W_p, b_p, convW_drug, convb_drug, convW_prot, convb_prot, vnW_drug, vnb_drug, vnW_prot, vnb_prot, vn_emb_drug, vn_emb_prot, drug_edge_index, drug_batch, drug_subgraph_node_index, drug_subgraph_indicator_index, prot_edge_index, prot_batch, prot_subgraph_node_index, prot_subgraph_indicator_index):
    raise NotImplementedError("write your pallas kernel here")


# speedup vs baseline: 1.0001x; 1.0001x over previous
import jax
import jax.numpy as jnp
from jax.experimental import pallas as pl

N = 50000
E = 800000
G = 128
H = 128
L = 3


def _segsum(data, ids, n):
    return jax.ops.segment_sum(data, ids, num_segments=n)


def _scatter_mean(data, ids, n):
    s = _segsum(data, ids, n)
    c = _segsum(jnp.ones((data.shape[0],), jnp.float32), ids, n)
    return s / jnp.maximum(c, 1.0)[:, None]


def _gcn(x, ei, W, b):
    src, dst = ei[0], ei[1]
    ones = jnp.ones((ei.shape[1],), jnp.float32)
    deg = _segsum(ones, dst, N) + 1.0
    dinv = jax.lax.rsqrt(deg)
    msg = x[src] * (dinv[src] * dinv[dst])[:, None]
    agg = _segsum(msg, dst, N) + x * (dinv * dinv)[:, None]
    return agg @ W + b


def _bn(x):
    m = jnp.mean(x, 0)
    v = jnp.var(x, 0)
    return (x - m) * jax.lax.rsqrt(v + 1e-5)


def _final_assemble_kernel(gd_ref, gp_ref, cd_ref, cp_ref, o_ref):
    gd = gd_ref[...]
    gp = gp_ref[...]
    cd = jnp.maximum(cd_ref[...], 1.0)
    cp = jnp.maximum(cp_ref[...], 1.0)
    o_ref[:, :H] = gd / cd
    o_ref[:, H:] = gp / cp


def kernel(drug_x, prot_x, d_feature, p_feature, W_denc, b_denc, W_penc, b_penc, W_d, b_d, W_p, b_p, convW_drug, convb_drug, convW_prot, convb_prot, vnW_drug, vnb_drug, vnW_prot, vnb_prot, vn_emb_drug, vn_emb_prot, drug_edge_index, drug_batch, drug_subgraph_node_index, drug_subgraph_indicator_index, prot_edge_index, prot_batch, prot_subgraph_node_index, prot_subgraph_indicator_index):
    h_d = drug_x @ W_denc + b_denc
    h_p = prot_x @ W_penc + b_penc
    vxd = jnp.broadcast_to(vn_emb_drug[0], (G, H))
    vxp = jnp.broadcast_to(vn_emb_prot[0], (G, H))
    fd = d_feature @ W_d + b_d
    fp = p_feature @ W_p + b_p
    for i in range(L):
        h_d = h_d + vxd[drug_batch] + fd[drug_batch]
        h_d = _gcn(h_d, drug_edge_index, convW_drug[i], convb_drug[i])
        h_p = h_p + vxp[prot_batch] + fp[prot_batch]
        h_p = _gcn(h_p, prot_edge_index, convW_prot[i], convb_prot[i])
        h_d = jax.nn.relu(_bn(h_d))
        h_p = jax.nn.relu(_bn(h_p))
        hds = _scatter_mean(h_d[drug_subgraph_node_index], drug_subgraph_indicator_index, N)
        hps = _scatter_mean(h_p[prot_subgraph_node_index], prot_subgraph_indicator_index, N)
        vxd = jax.nn.relu((_segsum(hds, drug_batch, G) + vxd) @ vnW_drug[i] + vnb_drug[i])
        vxp = jax.nn.relu((_segsum(hps, prot_batch, G) + vxp) @ vnW_prot[i] + vnb_prot[i])
    ones_n = jnp.ones((N,), jnp.float32)
    gd_s = _segsum(h_d, drug_batch, G)
    cd = _segsum(ones_n, drug_batch, G)
    gp_s = _segsum(h_p, prot_batch, G)
    cp = _segsum(ones_n, prot_batch, G)
    out = pl.pallas_call(
        _final_assemble_kernel,
        out_shape=jax.ShapeDtypeStruct((G, 2 * H), jnp.float32),
    )(gd_s, gp_s, cd[:, None], cp[:, None])
    return out


# speedup vs baseline: 2.0297x; 2.0294x over previous
import functools

import jax
import jax.numpy as jnp
from jax import lax
from jax.experimental import pallas as pl
from jax.experimental.pallas import tpu as pltpu
from jax.experimental.pallas import tpu_sc as plsc

N = 50000
E = 800000
G = 128
H = 128
L = 3

NC = 2
NS = 16
LN = 16
NW = NC * NS

M2 = 2 * N
RB = 400
NB = N // RB
R128 = 12544
P128 = 4
MP = P128 * NC * R128
R16 = MP // 2
K2E = 2 * E
KB = MP


def _enc_body(x_ref, w_ref, b_ref, o_ref):
    o_ref[...] = jnp.dot(x_ref[0], w_ref[0],
                         preferred_element_type=jnp.float32) + b_ref[0]


def _encode(xs, ws, bs):
    return pl.pallas_call(
        _enc_body,
        grid=(2, NB),
        in_specs=[
            pl.BlockSpec((1, RB, 80), lambda t, b: (t, b, 0)),
            pl.BlockSpec((1, 80, H), lambda t, b: (t, 0, 0)),
            pl.BlockSpec((1, 1, H), lambda t, b: (t, 0, 0)),
        ],
        out_specs=pl.BlockSpec((RB, H), lambda t, b: (t * NB + b, 0)),
        out_shape=jax.ShapeDtypeStruct((MP, H), jnp.float32),
    )(xs, ws, bs)


def _feat_body(f_ref, w_ref, b_ref, o_ref):
    o_ref[...] = (jnp.dot(f_ref[0], w_ref[0],
                          preferred_element_type=jnp.float32)
                  + b_ref[0])[None]


def _feat(fs, ws, bs):
    return pl.pallas_call(
        _feat_body,
        grid=(2,),
        in_specs=[
            pl.BlockSpec((1, G, 256), lambda t: (t, 0, 0)),
            pl.BlockSpec((1, 256, H), lambda t: (t, 0, 0)),
            pl.BlockSpec((1, 1, H), lambda t: (t, 0, 0)),
        ],
        out_specs=pl.BlockSpec((1, G, H), lambda t: (t, 0, 0)),
        out_shape=jax.ShapeDtypeStruct((2, G, H), jnp.float32),
    )(fs, ws, bs)


def _scalars_body(cd_ref, cs_ref, dinv_ref, invdeg_ref, invsub_ref):
    deg = jnp.sum(cd_ref[...], 0)[:, None] + 1.0
    dinv_ref[...] = lax.rsqrt(deg)
    invdeg_ref[...] = 1.0 / deg
    invsub_ref[...] = 1.0 / jnp.maximum(jnp.sum(cs_ref[...], 0)[:, None], 1.0)


def _scalars(cnt_dst, cnt_sub):
    o = jax.ShapeDtypeStruct((MP, 1), jnp.float32)
    SB = 2048
    return pl.pallas_call(
        _scalars_body,
        grid=(MP // SB,),
        in_specs=[pl.BlockSpec((NW, SB), lambda i: (0, i))] * 2,
        out_specs=[pl.BlockSpec((SB, 1), lambda i: (i, 0))] * 3,
        out_shape=[o, o, o],
    )(cnt_dst, cnt_sub)


def _xprep_body(h_ref, vxf_ref, bat_ref, dinv_ref, x_ref, u_ref):
    oh = (bat_ref[...] == lax.broadcasted_iota(jnp.int32, (1, G), 1)
          ).astype(jnp.float32)
    x = h_ref[...] + jnp.dot(oh, vxf_ref[0],
                             preferred_element_type=jnp.float32)
    x_ref[...] = x
    u_ref[...] = x * dinv_ref[...]


def _xprep(hn, vxf, bat2d, dinv):
    o = jax.ShapeDtypeStruct((MP, H), jnp.float32)
    return pl.pallas_call(
        _xprep_body,
        grid=(2, NB),
        in_specs=[
            pl.BlockSpec((RB, H), lambda t, b: (t * NB + b, 0)),
            pl.BlockSpec((1, G, H), lambda t, b: (t, 0, 0)),
            pl.BlockSpec((RB, 1), lambda t, b: (t * NB + b, 0)),
            pl.BlockSpec((RB, 1), lambda t, b: (t * NB + b, 0)),
        ],
        out_specs=[pl.BlockSpec((RB, H), lambda t, b: (t * NB + b, 0))] * 2,
        out_shape=[o, o],
    )(hn, vxf, bat2d, dinv)


def _mmbn_body(agg_ref, x_ref, dinv_ref, invdeg_ref, w_ref, b_ref,
               h1_ref, sum_ref, sq_ref):
    b = pl.program_id(1)
    pre = agg_ref[...] * dinv_ref[...] + x_ref[...] * invdeg_ref[...]
    h1 = jnp.dot(pre, w_ref[0], preferred_element_type=jnp.float32) + b_ref[0]
    h1_ref[...] = h1

    @pl.when(b == 0)
    def _():
        sum_ref[...] = jnp.zeros_like(sum_ref)
        sq_ref[...] = jnp.zeros_like(sq_ref)

    sum_ref[...] += jnp.sum(h1, 0)[None, None]
    sq_ref[...] += jnp.sum(h1 * h1, 0)[None, None]


def _mmbn(aggr, x, dinv, invdeg, w, bconv):
    return pl.pallas_call(
        _mmbn_body,
        grid=(2, NB),
        in_specs=[
            pl.BlockSpec((RB, H), lambda t, b: (t * NB + b, 0)),
            pl.BlockSpec((RB, H), lambda t, b: (t * NB + b, 0)),
            pl.BlockSpec((RB, 1), lambda t, b: (t * NB + b, 0)),
            pl.BlockSpec((RB, 1), lambda t, b: (t * NB + b, 0)),
            pl.BlockSpec((1, H, H), lambda t, b: (t, 0, 0)),
            pl.BlockSpec((1, 1, H), lambda t, b: (t, 0, 0)),
        ],
        out_specs=[
            pl.BlockSpec((RB, H), lambda t, b: (t * NB + b, 0)),
            pl.BlockSpec((1, 1, H), lambda t, b: (t, 0, 0)),
            pl.BlockSpec((1, 1, H), lambda t, b: (t, 0, 0)),
        ],
        out_shape=[
            jax.ShapeDtypeStruct((MP, H), jnp.float32),
            jax.ShapeDtypeStruct((2, 1, H), jnp.float32),
            jax.ShapeDtypeStruct((2, 1, H), jnp.float32),
        ],
    )(aggr, x, dinv, invdeg, w, bconv)


def _bnrelu_body(h1_ref, sum_ref, sq_ref, o_ref):
    m = sum_ref[0] * (1.0 / N)
    v = sq_ref[0] * (1.0 / N) - m * m
    o_ref[...] = jnp.maximum((h1_ref[...] - m) * lax.rsqrt(v + 1e-5), 0.0)


def _bnrelu(h1, s, sq):
    return pl.pallas_call(
        _bnrelu_body,
        grid=(2, NB),
        in_specs=[
            pl.BlockSpec((RB, H), lambda t, b: (t * NB + b, 0)),
            pl.BlockSpec((1, 1, H), lambda t, b: (t, 0, 0)),
            pl.BlockSpec((1, 1, H), lambda t, b: (t, 0, 0)),
        ],
        out_specs=pl.BlockSpec((RB, H), lambda t, b: (t * NB + b, 0)),
        out_shape=jax.ShapeDtypeStruct((MP, H), jnp.float32),
    )(h1, s, sq)


def _taccum_body(s_ref, scale_ref, bat_ref, t_ref, cnt_ref):
    b = pl.program_id(1)

    @pl.when(b == 0)
    def _():
        t_ref[...] = jnp.zeros_like(t_ref)
        cnt_ref[...] = jnp.zeros_like(cnt_ref)

    oh = (bat_ref[...] == lax.broadcasted_iota(jnp.int32, (1, G), 1)
          ).astype(jnp.float32)
    v = s_ref[...] * scale_ref[...]
    t_ref[...] += lax.dot_general(
        oh, v, (((0,), (0,)), ((), ())),
        preferred_element_type=jnp.float32)[None]
    cnt_ref[...] += jnp.sum(oh, 0)[None, None]


def _taccum(s, scale, bat2d):
    return pl.pallas_call(
        _taccum_body,
        grid=(2, NB),
        in_specs=[
            pl.BlockSpec((RB, H), lambda t, b: (t * NB + b, 0)),
            pl.BlockSpec((RB, 1), lambda t, b: (t * NB + b, 0)),
            pl.BlockSpec((RB, 1), lambda t, b: (t * NB + b, 0)),
        ],
        out_specs=[
            pl.BlockSpec((1, G, H), lambda t, b: (t, 0, 0)),
            pl.BlockSpec((1, 1, G), lambda t, b: (t, 0, 0)),
        ],
        out_shape=[
            jax.ShapeDtypeStruct((2, G, H), jnp.float32),
            jax.ShapeDtypeStruct((2, 1, G), jnp.float32),
        ],
    )(s, scale, bat2d)


def _vnupd_body(t_ref, vx_ref, w_ref, b_ref, o_ref):
    z = jnp.dot(t_ref[0] + vx_ref[0], w_ref[0],
                preferred_element_type=jnp.float32) + b_ref[0]
    o_ref[...] = jnp.maximum(z, 0.0)[None]


def _vnupd(t, vx, w, b):
    return pl.pallas_call(
        _vnupd_body,
        grid=(2,),
        in_specs=[
            pl.BlockSpec((1, G, H), lambda t_: (t_, 0, 0)),
            pl.BlockSpec((1, G, H), lambda t_: (t_, 0, 0)),
            pl.BlockSpec((1, H, H), lambda t_: (t_, 0, 0)),
            pl.BlockSpec((1, 1, H), lambda t_: (t_, 0, 0)),
        ],
        out_specs=pl.BlockSpec((1, G, H), lambda t_: (t_, 0, 0)),
        out_shape=jax.ShapeDtypeStruct((2, G, H), jnp.float32),
    )(t, vx, w, b)


def _assemble_body(t2_ref, cb_ref, o_ref):
    inv = 1.0 / jnp.maximum(cb_ref[0, 0][:, None], 1.0)
    o_ref[...] = t2_ref[0] * inv


def _assemble(t2, cntb):
    return pl.pallas_call(
        _assemble_body,
        grid=(2,),
        in_specs=[
            pl.BlockSpec((1, G, H), lambda t: (t, 0, 0)),
            pl.BlockSpec((1, 1, G), lambda t: (t, 0, 0)),
        ],
        out_specs=pl.BlockSpec((G, H), lambda t: (0, t)),
        out_shape=jax.ShapeDtypeStruct((G, 2 * H), jnp.float32),
    )(t2, cntb)


_GD = jax.lax.GatherDimensionNumbers(
    offset_dims=(), collapsed_slice_dims=(0,), start_index_map=(0,))


def _cumsum16(x):
    io = lax.iota(jnp.int32, LN)
    for k in (1, 2, 4, 8):
        idx = jnp.maximum(io - k, 0)
        y = lax.gather(x, idx[:, None], _GD, (1,),
                       mode=lax.GatherScatterMode.PROMISE_IN_BOUNDS)
        x = x + jnp.where(io >= k, y, jnp.int32(0))
    return x


def _make_scatter_rows(K, CH, W, P, R, sorted_ids, gather=True):
    NCHUNK = K // (NS * CH)
    assert NCHUNK * NS * CH == K and CH % LN == 0
    NG = CH // LN
    SLAB = R // NS
    B = 128
    STG = B + 2 * LN
    ZR = 16 if W == 128 else 64
    assert SLAB % ZR == 0
    OUTR = P * NC * R
    mesh = plsc.VectorSubcoreMesh(core_axis_name="c", subcore_axis_name="s",
                                  num_cores=NC, num_subcores=NS)

    scratch = [
        pltpu.VMEM((CH,), jnp.int32),
        pltpu.VMEM((STG,), jnp.int32),
        pltpu.VMEM((B,), jnp.int32),
        pltpu.VMEM((B, W), jnp.float32),
        pltpu.VMEM((ZR, W), jnp.float32),
        pltpu.VMEM_SHARED((R + 8, W), jnp.float32),
    ]
    if gather:
        scratch += [
            pltpu.VMEM((CH,), jnp.int32),
            pltpu.VMEM((STG,), jnp.int32),
            pltpu.VMEM((B,), jnp.int32),
            pltpu.SemaphoreType.DMA,
        ]

    @functools.partial(
        pl.kernel,
        out_type=jax.ShapeDtypeStruct((OUTR, W), jnp.float32),
        mesh=mesh,
        compiler_params=pltpu.CompilerParams(needs_layout_passes=False),
        scratch_types=scratch,
    )
    def scat(*refs):
        if gather:
            (table, src_h, dst_h, out, dstb, stg_d, fdst, rows, zb, acc,
             srcb, stg_s, fsrc, sem) = refs
        else:
            dst_h, out, dstb, stg_d, fdst, rows, zb, acc = refs
        c = lax.axis_index("c")
        s = lax.axis_index("s")
        zeros16 = jnp.zeros((LN,), jnp.float32)
        ones16 = jnp.ones((LN,), jnp.float32)
        trash = jnp.int32(R)

        def zrow(i, carry):
            for l in range(W // LN):
                zb[i, pl.ds(l * LN, LN)] = zeros16
            return carry
        lax.fori_loop(0, ZR, zrow, 0)

        if not gather:
            def onerow(i, carry):
                for l in range(W // LN):
                    rows[i, pl.ds(l * LN, LN)] = ones16
                return carry
            lax.fori_loop(0, B, onerow, 0)

        def do_flush(nvalid):
            for j in range(B // LN):
                idx = lax.iota(jnp.int32, LN) + (j * LN)
                valid = idx < nvalid
                dv = stg_d[pl.ds(j * LN, LN)]
                fdst[pl.ds(j * LN, LN)] = jnp.where(valid, dv, trash)
                if gather:
                    sv = stg_s[pl.ds(j * LN, LN)]
                    fsrc[pl.ds(j * LN, LN)] = jnp.where(valid, sv, 0)
            if gather:
                pltpu.async_copy(table.at[fsrc], rows, sem).wait()
            pltpu.sync_copy(rows, acc.at[fdst], add=True)

        for p in range(P):
            lo = jnp.int32(p * NC * R) + c * R
            hi = lo + R
            def zslab(z, carry):
                pltpu.sync_copy(zb, acc.at[pl.ds(s * SLAB + z * ZR, ZR)])
                return carry
            lax.fori_loop(0, SLAB // ZR, zslab, 0)
            plsc.subcore_barrier()

            def group(g, mcnt):
                dv = dstb[pl.ds(g * LN, LN)]
                m = (dv >= lo) & (dv < hi)
                cs = _cumsum16(jnp.where(m, jnp.int32(1), jnp.int32(0)))
                pos = jnp.where(m, mcnt + cs - 1, jnp.int32(STG - 1))
                if gather:
                    sv = srcb[pl.ds(g * LN, LN)]
                    plsc.store_scatter(stg_s, [pos], sv)
                plsc.store_scatter(stg_d, [pos], dv - lo)
                mcnt = mcnt + cs[LN - 1]

                def flush_rem(_):
                    do_flush(jnp.int32(B))
                    if gather:
                        stg_s[pl.ds(0, LN)] = stg_s[pl.ds(B, LN)]
                    stg_d[pl.ds(0, LN)] = stg_d[pl.ds(B, LN)]
                    return mcnt - B
                return lax.cond(mcnt >= B, flush_rem, lambda _: mcnt, 0)

            def chunk(ch, mcnt):
                off = (s * NCHUNK + ch) * CH
                pltpu.sync_copy(dst_h.at[pl.ds(off, CH)], dstb)
                if gather:
                    pltpu.sync_copy(src_h.at[pl.ds(off, CH)], srcb)
                if sorted_ids:
                    cmin = dstb[pl.ds(0, LN)][0]
                    cmax = dstb[pl.ds(CH - LN, LN)][LN - 1]
                    return lax.cond((cmax >= lo) & (cmin < hi),
                                    lambda a: lax.fori_loop(0, NG, group, a),
                                    lambda a: a, mcnt)
                return lax.fori_loop(0, NG, group, mcnt)

            mcnt = lax.fori_loop(0, NCHUNK, chunk, jnp.int32(0))

            def tail(_):
                do_flush(mcnt)
                return 0
            lax.cond(mcnt > 0, tail, lambda _: 0, 0)

            plsc.subcore_barrier()
            pltpu.sync_copy(acc.at[pl.ds(s * SLAB, SLAB)],
                            out.at[pl.ds(lo + s * SLAB, SLAB)])
            if p != P - 1:
                plsc.subcore_barrier()

    return scat


_scatter_builder = functools.lru_cache(maxsize=None)(_make_scatter_rows)


def _scat_gcn(t, s, d):
    return _scatter_builder(K2E, 2000, H, P128, R128, False)(t, s, d)


def _scat_sub(t, s, d):
    return _scatter_builder(K2E, 2000, H, P128, R128, True)(t, s, d)


def _make_hist(K, CH):
    NCHUNK = K // (NW * CH)
    assert NCHUNK * NW * CH == K and CH % LN == 0
    NG = CH // LN
    AR = MP + LN
    mesh = plsc.VectorSubcoreMesh(core_axis_name="c", subcore_axis_name="s",
                                  num_cores=NC, num_subcores=NS)

    @functools.partial(
        pl.kernel,
        out_type=jax.ShapeDtypeStruct((NW * MP,), jnp.float32),
        mesh=mesh,
        compiler_params=pltpu.CompilerParams(needs_layout_passes=False),
        scratch_types=[
            pltpu.VMEM((CH,), jnp.int32),
            pltpu.VMEM((AR,), jnp.float32),
        ],
    )
    def hist(idx_h, out, idxb, acc):
        c = lax.axis_index("c")
        s = lax.axis_index("s")
        wid = c * NS + s
        zeros16 = jnp.zeros((LN,), jnp.float32)
        ones16 = jnp.ones((LN,), jnp.float32)

        def zrow(i, carry):
            acc[pl.ds(i * LN, LN)] = zeros16
            return carry
        lax.fori_loop(0, AR // LN, zrow, 0)

        def group(g, carry):
            dv = idxb[pl.ds(g * LN, LN)]
            ok = (dv >= 0) & (dv < MP)
            iv = jnp.where(ok, dv, jnp.int32(MP))
            plsc.addupdate_scatter(acc, [iv], ones16)
            return carry

        def chunk(ch, carry):
            off = (wid * NCHUNK + ch) * CH
            pltpu.sync_copy(idx_h.at[pl.ds(off, CH)], idxb)
            return lax.fori_loop(0, NG, group, carry)

        lax.fori_loop(0, NCHUNK, chunk, 0)
        pltpu.sync_copy(acc.at[pl.ds(0, MP)], out.at[pl.ds(wid * MP, MP)])

    return hist


_hist_builder = functools.lru_cache(maxsize=None)(_make_hist)


def _hist_edge(d):
    return _hist_builder(K2E, 2000)(d)


def kernel(drug_x, prot_x, d_feature, p_feature, W_denc, b_denc, W_penc,
           b_penc, W_d, b_d, W_p, b_p, convW_drug, convb_drug, convW_prot,
           convb_prot, vnW_drug, vnb_drug, vnW_prot, vnb_prot, vn_emb_drug,
           vn_emb_prot, drug_edge_index, drug_batch,
           drug_subgraph_node_index, drug_subgraph_indicator_index,
           prot_edge_index, prot_batch, prot_subgraph_node_index,
           prot_subgraph_indicator_index):
    f32 = jnp.float32

    xs = jnp.stack([
        jnp.pad(drug_x, ((0, 0), (0, 2))),
        jnp.pad(prot_x, ((0, 0), (0, 10))),
    ])
    wenc = jnp.stack([
        jnp.pad(W_denc, ((0, 2), (0, 0))),
        jnp.pad(W_penc, ((0, 10), (0, 0))),
    ])
    benc = jnp.stack([b_denc[None], b_penc[None]])
    fs = jnp.stack([d_feature, p_feature])
    wf = jnp.stack([W_d, W_p])
    bf = jnp.stack([b_d[None], b_p[None]])
    convw = jnp.stack([convW_drug, convW_prot], 1)
    convb = jnp.stack([convb_drug[:, None], convb_prot[:, None]], 1)
    vnw = jnp.stack([vnW_drug, vnW_prot], 1)
    vnb = jnp.stack([vnb_drug[:, None], vnb_prot[:, None]], 1)

    src_all = jnp.concatenate([drug_edge_index[0], prot_edge_index[0] + N])
    dst_all = jnp.concatenate([drug_edge_index[1], prot_edge_index[1] + N])
    subn_all = jnp.concatenate([drug_subgraph_node_index,
                                prot_subgraph_node_index + N])
    subi_all = jnp.concatenate([drug_subgraph_indicator_index,
                                prot_subgraph_indicator_index + N])
    batch_all = jnp.concatenate([drug_batch, prot_batch + G])
    bat2d = batch_all.astype(jnp.int32)[:, None] - (
        jnp.arange(2, dtype=jnp.int32)[:, None]
        .repeat(N, 1).reshape(-1) * G)[:, None]
    ones_col = jnp.ones((MP, 1), f32)

    cnt_dst = _hist_edge(dst_all).reshape(NW, MP)
    cnt_sub = _hist_edge(subi_all).reshape(NW, MP)
    dinv, invdeg, invsub = _scalars(cnt_dst, cnt_sub)

    hn = _encode(xs, wenc, benc)
    f = _feat(fs, wf, bf)
    vx = jnp.stack([jnp.broadcast_to(vn_emb_drug[0], (G, H)),
                    jnp.broadcast_to(vn_emb_prot[0], (G, H))])

    for i in range(L):
        x, u = _xprep(hn, vx + f, bat2d, dinv)
        aggr = _scat_gcn(u, src_all, dst_all)
        h1, sm, sq = _mmbn(aggr, x, dinv, invdeg, convw[i], convb[i])
        hn = _bnrelu(h1, sm, sq)
        s = _scat_sub(hn, subn_all, subi_all)
        t, _ = _taccum(s, invsub, bat2d)
        vx = _vnupd(t, vx, vnw[i], vnb[i])

    t2, cnt2 = _taccum(hn, ones_col, bat2d)
    return _assemble(t2, cnt2)
